# baseline (speedup 1.0000x reference)
import functools

import jax
import jax.numpy as jnp
from jax import lax
from jax.experimental import pallas as pl
from jax.experimental.pallas import tpu as pltpu
from jax.experimental.pallas import tpu_sc as plsc

N_PROP = 256
M_GRID = 32
G = N_PROP * M_GRID
K_PTS = 4096
C_FEAT = 128
NS_A = 16
NS_B = 32
R2_A = 0.8 * 0.8
R2_B = 1.6 * 1.6
F32 = jnp.float32
BF16 = jnp.bfloat16


def _prep_body(grid_ref, kp_ref, feats_ref, w0a_ref, b0a_ref, w0b_ref,
               b0b_ref, qab_ref, da_ref, db_ref):
    kp = kp_ref[...]
    feats = feats_ref[...]
    w0a = w0a_ref[...]
    w0b = w0b_ref[...]
    qab_ref[:, 0:64] = (jnp.dot(kp, w0a[0:3], preferred_element_type=F32, precision=lax.Precision.HIGHEST)
                        + jnp.dot(feats, w0a[3:131], preferred_element_type=F32, precision=lax.Precision.HIGHEST)
                        + b0a_ref[...])
    qab_ref[:, 64:128] = (jnp.dot(kp, w0b[0:3], preferred_element_type=F32, precision=lax.Precision.HIGHEST)
                          + jnp.dot(feats, w0b[3:131], preferred_element_type=F32, precision=lax.Precision.HIGHEST)
                          + b0b_ref[...])
    rx = grid_ref[:, 0:1]
    ry = grid_ref[:, 1:2]
    rz = grid_ref[:, 2:3]
    da_ref[...] = -(rx * w0a[0][None, :] + ry * w0a[1][None, :]
                    + rz * w0a[2][None, :])
    db_ref[...] = -(rx * w0b[0][None, :] + ry * w0b[1][None, :]
                    + rz * w0b[2][None, :])


def _prep(grid, kp, feats_t, w0a, b0a2, w0b, b0b2):
    return pl.pallas_call(
        _prep_body,
        out_shape=[
            jax.ShapeDtypeStruct((K_PTS, 128), F32),
            jax.ShapeDtypeStruct((G, 64), F32),
            jax.ShapeDtypeStruct((G, 64), F32),
        ],
    )(grid, kp, feats_t, w0a, b0a2, w0b, b0b2)


_BG = 128


def _select_body(gm3_ref, g2_ref, kpt_ref, k2_ref, idxa_ref, idxb_ref):
    bg = _BG
    d2 = (lax.dot_general(gm3_ref[...], kpt_ref[...],
                          (((0,), (0,)), ((), ())),
                          preferred_element_type=F32)
          + jnp.transpose(g2_ref[...]) + k2_ref[...])
    ii = lax.broadcasted_iota(jnp.int32, (128, 128), 0)
    jj = lax.broadcasted_iota(jnp.int32, (128, 128), 1)
    l128 = (ii <= jj).astype(BF16)
    s32 = (ii[0:32, 0:32] < jj[0:32, 0:32]).astype(BF16)

    def cums(mask):
        m2 = mask.astype(BF16).reshape(bg * 32, 128)
        intra = jnp.dot(m2, l128, preferred_element_type=F32)
        intra = intra.reshape(bg, 32, 128)
        totals = intra[:, :, 127]
        off = jnp.dot(totals.astype(BF16), s32, preferred_element_type=F32)
        c = (intra + off[:, :, None]).reshape(bg, K_PTS)
        cnt = (totals + off[:, 31:32])[:, 31:32]
        return c, cnt

    def indices(c, cnt, ns):
        cols = [jnp.sum((c <= float(j)).astype(F32), axis=1, keepdims=True)
                for j in range(ns)]
        raw = jnp.concatenate(cols, axis=1)
        first = jnp.where(cnt > 0.0, raw[:, 0:1], 0.0)
        jcol = lax.broadcasted_iota(jnp.int32, (bg, ns), 1).astype(F32)
        return jnp.where(jcol < cnt, raw, first).astype(jnp.int32)

    ca, cnta = cums(d2 < R2_A)
    cb, cntb = cums(d2 < R2_B)
    idxa_ref[...] = indices(ca, cnta, NS_A)
    idxb_ref[...] = indices(cb, cntb, NS_B)


def _select(gm3, g2, kpt, k2):
    return pl.pallas_call(
        _select_body,
        grid=(G // _BG,),
        in_specs=[
            pl.BlockSpec((3, _BG), lambda i: (0, i)),
            pl.BlockSpec((1, _BG), lambda i: (0, i)),
            pl.BlockSpec((3, K_PTS), lambda i: (0, 0)),
            pl.BlockSpec((1, K_PTS), lambda i: (0, 0)),
        ],
        out_specs=[
            pl.BlockSpec((_BG, NS_A), lambda i: (i, 0)),
            pl.BlockSpec((_BG, NS_B), lambda i: (i, 0)),
        ],
        out_shape=[
            jax.ShapeDtypeStruct((G, NS_A), jnp.int32),
            jax.ShapeDtypeStruct((G, NS_B), jnp.int32),
        ],
    )(gm3, g2, kpt, k2)


_NBUF = 4


def _make_sc_gather():
    info = plsc.get_sparse_core_info()
    nc, nsc = info.num_cores, info.num_subcores
    nw = nc * nsc
    na = G * NS_A // nw
    nb = G * NS_B // nw
    mesh = plsc.VectorSubcoreMesh(core_axis_name="c", subcore_axis_name="s")

    @functools.partial(
        pl.kernel,
        out_type=[
            jax.ShapeDtypeStruct((G * NS_A, 128), F32),
            jax.ShapeDtypeStruct((G * NS_B, 128), F32),
        ],
        mesh=mesh,
        scratch_types=(
            [pltpu.VMEM((128,), jnp.int32) for _ in range(_NBUF)]
            + [pltpu.VMEM((128, 128), F32) for _ in range(_NBUF)]
            + [pltpu.SemaphoreType.DMA for _ in range(3 * _NBUF)]
        ),
    )
    def sc_gather(idxa_hbm, idxb_hbm, qab_hbm, ga_hbm, gb_hbm, *scratch):
        gidx = scratch[:_NBUF]
        rows = scratch[_NBUF:2 * _NBUF]
        si = scratch[2 * _NBUF:3 * _NBUF]
        sg = scratch[3 * _NBUF:4 * _NBUF]
        so = scratch[4 * _NBUF:5 * _NBUF]
        wid = lax.axis_index("s") * nc + lax.axis_index("c")

        def run(idx_hbm, out_hbm, base, nchunk):
            def grp(g, _):
                off = base + g * (128 * _NBUF)
                cis = [pltpu.async_copy(idx_hbm.at[pl.ds(off + t * 128, 128)],
                                        gidx[t], si[t])
                       for t in range(_NBUF)]
                cgs = []
                for t in range(_NBUF):
                    cis[t].wait()
                    cgs.append(pltpu.async_copy(qab_hbm.at[gidx[t]], rows[t],
                                                sg[t]))
                cos = []
                for t in range(_NBUF):
                    cgs[t].wait()
                    cos.append(pltpu.async_copy(
                        rows[t], out_hbm.at[pl.ds(off + t * 128, 128)], so[t]))
                for t in range(_NBUF):
                    cos[t].wait()
                return 0

            lax.fori_loop(0, nchunk // _NBUF, grp, 0)

        run(idxa_hbm, ga_hbm, wid * na, na // 128)
        run(idxb_hbm, gb_hbm, wid * nb, nb // 128)

    return sc_gather


_SC_GATHER = _make_sc_gather()


def _mlp_body(ns, g_ref, d_ref, w1_ref, b1_ref, out_ref, half):
    bg = g_ref.shape[0]
    x = g_ref[...][:, :, 64 * half:64 * half + 64]
    h0 = jnp.maximum(x + d_ref[...][:, None, :], 0.0)
    h1 = jnp.dot(h0.reshape(bg * ns, 64), w1_ref[...],
                 preferred_element_type=F32, precision=lax.Precision.HIGHEST) + b1_ref[...]
    h1 = jnp.maximum(h1, 0.0)
    out_ref[...] = jnp.max(h1.reshape(bg, ns, 64), axis=1)


def _mlp_max(gathered, d, w1, b12, ns, half):
    bg = 256
    return pl.pallas_call(
        functools.partial(_mlp_body, ns, half=half),
        grid=(G // bg,),
        in_specs=[
            pl.BlockSpec((bg, ns, 128), lambda i: (i, 0, 0)),
            pl.BlockSpec((bg, 64), lambda i: (i, 0)),
            pl.BlockSpec((64, 64), lambda i: (0, 0)),
            pl.BlockSpec((1, 64), lambda i: (0, 0)),
        ],
        out_specs=pl.BlockSpec((bg, 64), lambda i: (i, 0)),
        out_shape=jax.ShapeDtypeStruct((G, 64), F32),
    )(gathered.reshape(G, ns, 128), d, w1, b12)


def _tail_body(pf_ref, w1_ref, b1_ref, w2_ref, b2_ref, out_ref):
    h = jnp.dot(pf_ref[...], w1_ref[...], preferred_element_type=F32, precision=lax.Precision.HIGHEST) + b1_ref[...]
    h = jnp.maximum(h, 0.0)
    o = jnp.dot(h, w2_ref[...], preferred_element_type=F32, precision=lax.Precision.HIGHEST) + b2_ref[...]
    out_ref[...] = jnp.maximum(o, 0.0)


def _tail(pf, wr1, br12, wr2, br22):
    return pl.pallas_call(
        _tail_body,
        out_shape=jax.ShapeDtypeStruct((N_PROP, 256), F32),
    )(pf, wr1, br12, wr2, br22)


def kernel(proposals_wlh, proposals_yaw, proposals_center, keypoints_xyz,
           keypoints_features, grid_noise, W0a, b0a, W1a, b1a, W0b, b0b,
           W1b, b1b, Wr1, br1, Wr2, br2):
    kp = keypoints_xyz[0]
    kpt = jnp.transpose(kp)
    feats_t = jnp.transpose(keypoints_features[0])
    gp = grid_noise * proposals_wlh[:, None, :]
    xy = gp[..., :2]
    z = gp[..., 2:]
    cth = jnp.cos(proposals_yaw)
    sth = jnp.sin(proposals_yaw)
    rot = jnp.stack([cth, -sth, sth, cth], axis=-1).reshape(-1, 2, 2)
    xy_r = jnp.einsum('ijk,imk->imj', rot, xy)
    grid = (jnp.concatenate([xy_r, z], axis=-1)
            + proposals_center[:, None, :]).reshape(G, 3)
    gm3 = jnp.transpose(-2.0 * grid)
    g2 = jnp.sum(grid * grid, axis=-1)[None, :]
    k2 = jnp.sum(kp * kp, axis=-1)[None, :]

    qab, da, db = _prep(grid, kp, feats_t,
                        W0a, b0a.reshape(1, 64), W0b, b0b.reshape(1, 64))

    idxa, idxb = _select(gm3, g2, kpt, k2)

    ga, gb = _SC_GATHER(idxa.reshape(G * NS_A), idxb.reshape(G * NS_B), qab)

    pa = _mlp_max(ga, da, W1a, b1a.reshape(1, 64), NS_A, 0)
    pb = _mlp_max(gb, db, W1b, b1b.reshape(1, 64), NS_B, 1)

    pooled = jnp.concatenate([pa, pb], axis=-1)
    pf = (pooled.reshape(N_PROP, M_GRID, 8, 16)
          .transpose(1, 2, 3, 0).reshape(N_PROP, M_GRID * C_FEAT))
    out = _tail(pf, Wr1, br1.reshape(1, 512), Wr2, br2.reshape(1, 256))
    return out.reshape(1, N_PROP, 256)


# speedup vs baseline: 1.3177x; 1.3177x over previous
import functools

import jax
import jax.numpy as jnp
from jax import lax
from jax.experimental import pallas as pl
from jax.experimental.pallas import tpu as pltpu
from jax.experimental.pallas import tpu_sc as plsc

N_PROP = 256
M_GRID = 32
G = N_PROP * M_GRID
K_PTS = 4096
C_FEAT = 128
NS_A = 16
NS_B = 32
R2_A = 0.8 * 0.8
R2_B = 1.6 * 1.6
F32 = jnp.float32
BF16 = jnp.bfloat16


def _prep_body(grid_ref, kp_ref, feats_ref, w0a_ref, b0a_ref, w0b_ref,
               b0b_ref, qab_ref, da_ref, db_ref):
    kp = kp_ref[...]
    feats = feats_ref[...]
    w0a = w0a_ref[...]
    w0b = w0b_ref[...]
    qab_ref[:, 0:64] = (jnp.dot(kp, w0a[0:3], preferred_element_type=F32, precision=lax.Precision.HIGHEST)
                        + jnp.dot(feats, w0a[3:131], preferred_element_type=F32, precision=lax.Precision.HIGHEST)
                        + b0a_ref[...])
    qab_ref[:, 64:128] = (jnp.dot(kp, w0b[0:3], preferred_element_type=F32, precision=lax.Precision.HIGHEST)
                          + jnp.dot(feats, w0b[3:131], preferred_element_type=F32, precision=lax.Precision.HIGHEST)
                          + b0b_ref[...])
    rx = grid_ref[:, 0:1]
    ry = grid_ref[:, 1:2]
    rz = grid_ref[:, 2:3]
    da_ref[...] = -(rx * w0a[0][None, :] + ry * w0a[1][None, :]
                    + rz * w0a[2][None, :])
    db_ref[...] = -(rx * w0b[0][None, :] + ry * w0b[1][None, :]
                    + rz * w0b[2][None, :])


def _prep(grid, kp, feats_t, w0a, b0a2, w0b, b0b2):
    return pl.pallas_call(
        _prep_body,
        out_shape=[
            jax.ShapeDtypeStruct((K_PTS, 128), F32),
            jax.ShapeDtypeStruct((G, 64), F32),
            jax.ShapeDtypeStruct((G, 64), F32),
        ],
    )(grid, kp, feats_t, w0a, b0a2, w0b, b0b2)


_BG = 128


def _select_body(gm3_ref, g2_ref, kpt_ref, k2_ref, idxa_ref, idxb_ref):
    bg = _BG
    d2 = (lax.dot_general(gm3_ref[...], kpt_ref[...],
                          (((0,), (0,)), ((), ())),
                          preferred_element_type=F32)
          + jnp.transpose(g2_ref[...]) + k2_ref[...])
    ii = lax.broadcasted_iota(jnp.int32, (128, 128), 0)
    jj = lax.broadcasted_iota(jnp.int32, (128, 128), 1)
    l128 = (ii <= jj).astype(BF16)
    s32 = (ii[0:32, 0:32] < jj[0:32, 0:32]).astype(BF16)

    def cums(mask):
        m2 = mask.astype(BF16).reshape(bg * 32, 128)
        intra = jnp.dot(m2, l128, preferred_element_type=F32)
        intra = intra.reshape(bg, 32, 128)
        totals = intra[:, :, 127]
        off = jnp.dot(totals.astype(BF16), s32, preferred_element_type=F32)
        c = (intra + off[:, :, None]).reshape(bg, K_PTS)
        cnt = (totals + off[:, 31:32])[:, 31:32]
        return c, cnt

    def indices(c, cnt, ns):
        cols = [jnp.sum((c <= float(j)).astype(F32), axis=1, keepdims=True)
                for j in range(ns)]
        raw = jnp.concatenate(cols, axis=1)
        first = jnp.where(cnt > 0.0, raw[:, 0:1], 0.0)
        jcol = lax.broadcasted_iota(jnp.int32, (bg, ns), 1).astype(F32)
        return jnp.where(jcol < cnt, raw, first).astype(jnp.int32)

    ca, cnta = cums(d2 < R2_A)
    cb, cntb = cums(d2 < R2_B)
    idxa_ref[...] = indices(ca, cnta, NS_A)
    idxb_ref[...] = indices(cb, cntb, NS_B)


def _select(gm3, g2, kpt, k2):
    return pl.pallas_call(
        _select_body,
        grid=(G // _BG,),
        in_specs=[
            pl.BlockSpec((3, _BG), lambda i: (0, i)),
            pl.BlockSpec((1, _BG), lambda i: (0, i)),
            pl.BlockSpec((3, K_PTS), lambda i: (0, 0)),
            pl.BlockSpec((1, K_PTS), lambda i: (0, 0)),
        ],
        out_specs=[
            pl.BlockSpec((_BG, NS_A), lambda i: (i, 0)),
            pl.BlockSpec((_BG, NS_B), lambda i: (i, 0)),
        ],
        out_shape=[
            jax.ShapeDtypeStruct((G, NS_A), jnp.int32),
            jax.ShapeDtypeStruct((G, NS_B), jnp.int32),
        ],
    )(gm3, g2, kpt, k2)


_NBUF = 4


def _make_sc_gather():
    info = plsc.get_sparse_core_info()
    nc, nsc = info.num_cores, info.num_subcores
    nw = nc * nsc
    na = G * NS_A // nw
    nb = G * NS_B // nw
    mesh = plsc.VectorSubcoreMesh(core_axis_name="c", subcore_axis_name="s")

    @functools.partial(
        pl.kernel,
        out_type=[
            jax.ShapeDtypeStruct((G * NS_A, 128), F32),
            jax.ShapeDtypeStruct((G * NS_B, 128), F32),
        ],
        mesh=mesh,
        scratch_types=(
            [pltpu.VMEM((128,), jnp.int32) for _ in range(_NBUF)]
            + [pltpu.VMEM((128, 128), F32) for _ in range(_NBUF)]
            + [pltpu.VMEM_SHARED((K_PTS, 128), F32)]
            + [pltpu.SemaphoreType.DMA for _ in range(3 * _NBUF)]
        ),
    )
    def sc_gather(idxa_hbm, idxb_hbm, qab_hbm, ga_hbm, gb_hbm, *scratch):
        gidx = scratch[:_NBUF]
        rows = scratch[_NBUF:2 * _NBUF]
        tabs = scratch[2 * _NBUF]
        si = scratch[2 * _NBUF + 1:3 * _NBUF + 1]
        sg = scratch[3 * _NBUF + 1:4 * _NBUF + 1]
        so = scratch[4 * _NBUF + 1:5 * _NBUF + 1]
        wid = lax.axis_index("s") * nc + lax.axis_index("c")

        @pl.when(lax.axis_index("s") == 0)
        def _():
            pltpu.sync_copy(qab_hbm, tabs)
        plsc.subcore_barrier()

        def run(idx_hbm, out_hbm, base, nchunk):
            def grp(g, _):
                off = base + g * (128 * _NBUF)
                cis = [pltpu.async_copy(idx_hbm.at[pl.ds(off + t * 128, 128)],
                                        gidx[t], si[t])
                       for t in range(_NBUF)]
                cgs = []
                for t in range(_NBUF):
                    cis[t].wait()
                    cgs.append(pltpu.async_copy(tabs.at[gidx[t]], rows[t],
                                                sg[t]))
                cos = []
                for t in range(_NBUF):
                    cgs[t].wait()
                    cos.append(pltpu.async_copy(
                        rows[t], out_hbm.at[pl.ds(off + t * 128, 128)], so[t]))
                for t in range(_NBUF):
                    cos[t].wait()
                return 0

            lax.fori_loop(0, nchunk // _NBUF, grp, 0)

        run(idxa_hbm, ga_hbm, wid * na, na // 128)
        run(idxb_hbm, gb_hbm, wid * nb, nb // 128)

    return sc_gather


_SC_GATHER = _make_sc_gather()


def _mlp_body(ns, g_ref, d_ref, w1_ref, b1_ref, out_ref, half):
    bg = g_ref.shape[0]
    x = g_ref[...][:, :, 64 * half:64 * half + 64]
    h0 = jnp.maximum(x + d_ref[...][:, None, :], 0.0)
    h1 = jnp.dot(h0.reshape(bg * ns, 64), w1_ref[...],
                 preferred_element_type=F32, precision=lax.Precision.HIGHEST) + b1_ref[...]
    h1 = jnp.maximum(h1, 0.0)
    out_ref[...] = jnp.max(h1.reshape(bg, ns, 64), axis=1)


def _mlp_max(gathered, d, w1, b12, ns, half):
    bg = 256
    return pl.pallas_call(
        functools.partial(_mlp_body, ns, half=half),
        grid=(G // bg,),
        in_specs=[
            pl.BlockSpec((bg, ns, 128), lambda i: (i, 0, 0)),
            pl.BlockSpec((bg, 64), lambda i: (i, 0)),
            pl.BlockSpec((64, 64), lambda i: (0, 0)),
            pl.BlockSpec((1, 64), lambda i: (0, 0)),
        ],
        out_specs=pl.BlockSpec((bg, 64), lambda i: (i, 0)),
        out_shape=jax.ShapeDtypeStruct((G, 64), F32),
    )(gathered.reshape(G, ns, 128), d, w1, b12)


def _tail_body(pf_ref, w1_ref, b1_ref, w2_ref, b2_ref, out_ref):
    h = jnp.dot(pf_ref[...], w1_ref[...], preferred_element_type=F32, precision=lax.Precision.HIGHEST) + b1_ref[...]
    h = jnp.maximum(h, 0.0)
    o = jnp.dot(h, w2_ref[...], preferred_element_type=F32, precision=lax.Precision.HIGHEST) + b2_ref[...]
    out_ref[...] = jnp.maximum(o, 0.0)


def _tail(pf, wr1, br12, wr2, br22):
    return pl.pallas_call(
        _tail_body,
        out_shape=jax.ShapeDtypeStruct((N_PROP, 256), F32),
    )(pf, wr1, br12, wr2, br22)


def kernel(proposals_wlh, proposals_yaw, proposals_center, keypoints_xyz,
           keypoints_features, grid_noise, W0a, b0a, W1a, b1a, W0b, b0b,
           W1b, b1b, Wr1, br1, Wr2, br2):
    kp = keypoints_xyz[0]
    kpt = jnp.transpose(kp)
    feats_t = jnp.transpose(keypoints_features[0])
    gp = grid_noise * proposals_wlh[:, None, :]
    xy = gp[..., :2]
    z = gp[..., 2:]
    cth = jnp.cos(proposals_yaw)
    sth = jnp.sin(proposals_yaw)
    rot = jnp.stack([cth, -sth, sth, cth], axis=-1).reshape(-1, 2, 2)
    xy_r = jnp.einsum('ijk,imk->imj', rot, xy)
    grid = (jnp.concatenate([xy_r, z], axis=-1)
            + proposals_center[:, None, :]).reshape(G, 3)
    gm3 = jnp.transpose(-2.0 * grid)
    g2 = jnp.sum(grid * grid, axis=-1)[None, :]
    k2 = jnp.sum(kp * kp, axis=-1)[None, :]

    qab, da, db = _prep(grid, kp, feats_t,
                        W0a, b0a.reshape(1, 64), W0b, b0b.reshape(1, 64))

    idxa, idxb = _select(gm3, g2, kpt, k2)

    ga, gb = _SC_GATHER(idxa.reshape(G * NS_A), idxb.reshape(G * NS_B), qab)

    pa = _mlp_max(ga, da, W1a, b1a.reshape(1, 64), NS_A, 0)
    pb = _mlp_max(gb, db, W1b, b1b.reshape(1, 64), NS_B, 1)

    pooled = jnp.concatenate([pa, pb], axis=-1)
    pf = (pooled.reshape(N_PROP, M_GRID, 8, 16)
          .transpose(1, 2, 3, 0).reshape(N_PROP, M_GRID * C_FEAT))
    out = _tail(pf, Wr1, br1.reshape(1, 512), Wr2, br2.reshape(1, 256))
    return out.reshape(1, N_PROP, 256)


# speedup vs baseline: 1.4449x; 1.0966x over previous
import functools

import jax
import jax.numpy as jnp
from jax import lax
from jax.experimental import pallas as pl
from jax.experimental.pallas import tpu as pltpu
from jax.experimental.pallas import tpu_sc as plsc

N_PROP = 256
M_GRID = 32
G = N_PROP * M_GRID
K_PTS = 4096
C_FEAT = 128
NS_A = 16
NS_B = 32
R2_A = 0.8 * 0.8
R2_B = 1.6 * 1.6
F32 = jnp.float32
BF16 = jnp.bfloat16


def _prep_body(grid_ref, kp_ref, feats_ref, w0a_ref, b0a_ref, w0b_ref,
               b0b_ref, qab_ref, da_ref, db_ref):
    kp = kp_ref[...]
    feats = feats_ref[...]
    w0a = w0a_ref[...]
    w0b = w0b_ref[...]
    qab_ref[:, 0:64] = (jnp.dot(kp, w0a[0:3], preferred_element_type=F32, precision=lax.Precision.HIGHEST)
                        + jnp.dot(feats, w0a[3:131], preferred_element_type=F32, precision=lax.Precision.HIGHEST)
                        + b0a_ref[...])
    qab_ref[:, 64:128] = (jnp.dot(kp, w0b[0:3], preferred_element_type=F32, precision=lax.Precision.HIGHEST)
                          + jnp.dot(feats, w0b[3:131], preferred_element_type=F32, precision=lax.Precision.HIGHEST)
                          + b0b_ref[...])
    rx = grid_ref[:, 0:1]
    ry = grid_ref[:, 1:2]
    rz = grid_ref[:, 2:3]
    da_ref[...] = -(rx * w0a[0][None, :] + ry * w0a[1][None, :]
                    + rz * w0a[2][None, :])
    db_ref[...] = -(rx * w0b[0][None, :] + ry * w0b[1][None, :]
                    + rz * w0b[2][None, :])


def _prep(grid, kp, feats_t, w0a, b0a2, w0b, b0b2):
    return pl.pallas_call(
        _prep_body,
        out_shape=[
            jax.ShapeDtypeStruct((K_PTS, 128), F32),
            jax.ShapeDtypeStruct((G, 64), F32),
            jax.ShapeDtypeStruct((G, 64), F32),
        ],
    )(grid, kp, feats_t, w0a, b0a2, w0b, b0b2)


_BG = 128


def _select_body(gm3_ref, g2_ref, kpt_ref, k2_ref, idxa_ref, idxb_ref):
    bg = _BG
    d2 = (lax.dot_general(gm3_ref[...], kpt_ref[...],
                          (((0,), (0,)), ((), ())),
                          preferred_element_type=F32)
          + jnp.transpose(g2_ref[...]) + k2_ref[...])
    ii = lax.broadcasted_iota(jnp.int32, (128, 128), 0)
    jj = lax.broadcasted_iota(jnp.int32, (128, 128), 1)
    l128 = (ii <= jj).astype(BF16)
    s32 = (ii[0:32, 0:32] < jj[0:32, 0:32]).astype(BF16)

    def cums(mask):
        m2 = mask.astype(BF16).reshape(bg * 32, 128)
        intra = jnp.dot(m2, l128, preferred_element_type=F32)
        intra = intra.reshape(bg, 32, 128)
        totals = intra[:, :, 127]
        off = jnp.dot(totals.astype(BF16), s32, preferred_element_type=F32)
        c = (intra + off[:, :, None]).reshape(bg, K_PTS)
        cnt = (totals + off[:, 31:32])[:, 31:32]
        return c, cnt

    def indices(c, cnt, ns):
        cols = [jnp.sum((c <= float(j)).astype(F32), axis=1, keepdims=True)
                for j in range(ns)]
        raw = jnp.concatenate(cols, axis=1)
        first = jnp.where(cnt > 0.0, raw[:, 0:1], 0.0)
        jcol = lax.broadcasted_iota(jnp.int32, (bg, ns), 1).astype(F32)
        return jnp.where(jcol < cnt, raw, first).astype(jnp.int32)

    ca, cnta = cums(d2 < R2_A)
    cb, cntb = cums(d2 < R2_B)
    idxa_ref[...] = indices(ca, cnta, NS_A)
    idxb_ref[...] = indices(cb, cntb, NS_B)


def _select(gm3, g2, kpt, k2):
    return pl.pallas_call(
        _select_body,
        grid=(G // _BG,),
        in_specs=[
            pl.BlockSpec((3, _BG), lambda i: (0, i)),
            pl.BlockSpec((1, _BG), lambda i: (0, i)),
            pl.BlockSpec((3, K_PTS), lambda i: (0, 0)),
            pl.BlockSpec((1, K_PTS), lambda i: (0, 0)),
        ],
        out_specs=[
            pl.BlockSpec((_BG, NS_A), lambda i: (i, 0)),
            pl.BlockSpec((_BG, NS_B), lambda i: (i, 0)),
        ],
        out_shape=[
            jax.ShapeDtypeStruct((G, NS_A), jnp.int32),
            jax.ShapeDtypeStruct((G, NS_B), jnp.int32),
        ],
    )(gm3, g2, kpt, k2)


_NBUF = 4


def _make_sc_gather():
    info = plsc.get_sparse_core_info()
    nc, nsc = info.num_cores, info.num_subcores
    nw = nc * nsc
    na = G * NS_A // nw
    nb = G * NS_B // nw
    mesh = plsc.VectorSubcoreMesh(core_axis_name="c", subcore_axis_name="s")

    @functools.partial(
        pl.kernel,
        out_type=[
            jax.ShapeDtypeStruct((G * NS_A, 128), F32),
            jax.ShapeDtypeStruct((G * NS_B, 128), F32),
        ],
        mesh=mesh,
        scratch_types=(
            [pltpu.VMEM((128,), jnp.int32) for _ in range(_NBUF)]
            + [pltpu.VMEM((128, 128), F32) for _ in range(_NBUF)]
            + [pltpu.VMEM_SHARED((K_PTS, 128), F32)]
            + [pltpu.SemaphoreType.DMA for _ in range(3 * _NBUF)]
        ),
    )
    def sc_gather(idxa_hbm, idxb_hbm, qab_hbm, ga_hbm, gb_hbm, *scratch):
        gidx = scratch[:_NBUF]
        rows = scratch[_NBUF:2 * _NBUF]
        tabs = scratch[2 * _NBUF]
        si = scratch[2 * _NBUF + 1:3 * _NBUF + 1]
        sg = scratch[3 * _NBUF + 1:4 * _NBUF + 1]
        so = scratch[4 * _NBUF + 1:5 * _NBUF + 1]
        wid = lax.axis_index("s") * nc + lax.axis_index("c")

        @pl.when(lax.axis_index("s") == 0)
        def _():
            pltpu.sync_copy(qab_hbm, tabs)
        plsc.subcore_barrier()

        def run(idx_hbm, out_hbm, base, nchunk):
            def grp(g, _):
                off = base + g * (128 * _NBUF)
                cis = [pltpu.async_copy(idx_hbm.at[pl.ds(off + t * 128, 128)],
                                        gidx[t], si[t])
                       for t in range(_NBUF)]
                cgs = []
                for t in range(_NBUF):
                    cis[t].wait()
                    cgs.append(pltpu.async_copy(tabs.at[gidx[t]], rows[t],
                                                sg[t]))
                cos = []
                for t in range(_NBUF):
                    cgs[t].wait()
                    cos.append(pltpu.async_copy(
                        rows[t], out_hbm.at[pl.ds(off + t * 128, 128)], so[t]))
                for t in range(_NBUF):
                    cos[t].wait()
                return 0

            lax.fori_loop(0, nchunk // _NBUF, grp, 0)

        run(idxa_hbm, ga_hbm, wid * na, na // 128)
        run(idxb_hbm, gb_hbm, wid * nb, nb // 128)

    return sc_gather


_SC_GATHER = _make_sc_gather()


def _mlp_body(ns, g_ref, d_ref, w1_ref, b1_ref, out_ref, half):
    bg = g_ref.shape[0]
    x = g_ref[...][:, :, 64 * half:64 * half + 64]
    h0 = jnp.maximum(x + d_ref[...][:, None, :], 0.0)
    h1 = jnp.dot(h0.reshape(bg * ns, 64), w1_ref[...],
                 preferred_element_type=F32) + b1_ref[...]
    h1 = jnp.maximum(h1, 0.0)
    out_ref[...] = jnp.max(h1.reshape(bg, ns, 64), axis=1)


def _mlp_max(gathered, d, w1, b12, ns, half):
    bg = 256
    return pl.pallas_call(
        functools.partial(_mlp_body, ns, half=half),
        grid=(G // bg,),
        in_specs=[
            pl.BlockSpec((bg, ns, 128), lambda i: (i, 0, 0)),
            pl.BlockSpec((bg, 64), lambda i: (i, 0)),
            pl.BlockSpec((64, 64), lambda i: (0, 0)),
            pl.BlockSpec((1, 64), lambda i: (0, 0)),
        ],
        out_specs=pl.BlockSpec((bg, 64), lambda i: (i, 0)),
        out_shape=jax.ShapeDtypeStruct((G, 64), F32),
    )(gathered.reshape(G, ns, 128), d, w1, b12)


def _tail_body(pf_ref, w1_ref, b1_ref, w2_ref, b2_ref, out_ref):
    h = jnp.dot(pf_ref[...], w1_ref[...], preferred_element_type=F32, precision=lax.Precision.HIGHEST) + b1_ref[...]
    h = jnp.maximum(h, 0.0)
    o = jnp.dot(h, w2_ref[...], preferred_element_type=F32, precision=lax.Precision.HIGHEST) + b2_ref[...]
    out_ref[...] = jnp.maximum(o, 0.0)


def _tail(pf, wr1, br12, wr2, br22):
    return pl.pallas_call(
        _tail_body,
        out_shape=jax.ShapeDtypeStruct((N_PROP, 256), F32),
    )(pf, wr1, br12, wr2, br22)


def kernel(proposals_wlh, proposals_yaw, proposals_center, keypoints_xyz,
           keypoints_features, grid_noise, W0a, b0a, W1a, b1a, W0b, b0b,
           W1b, b1b, Wr1, br1, Wr2, br2):
    kp = keypoints_xyz[0]
    kpt = jnp.transpose(kp)
    feats_t = jnp.transpose(keypoints_features[0])
    gp = grid_noise * proposals_wlh[:, None, :]
    xy = gp[..., :2]
    z = gp[..., 2:]
    cth = jnp.cos(proposals_yaw)
    sth = jnp.sin(proposals_yaw)
    rot = jnp.stack([cth, -sth, sth, cth], axis=-1).reshape(-1, 2, 2)
    xy_r = jnp.einsum('ijk,imk->imj', rot, xy)
    grid = (jnp.concatenate([xy_r, z], axis=-1)
            + proposals_center[:, None, :]).reshape(G, 3)
    gm3 = jnp.transpose(-2.0 * grid)
    g2 = jnp.sum(grid * grid, axis=-1)[None, :]
    k2 = jnp.sum(kp * kp, axis=-1)[None, :]

    qab, da, db = _prep(grid, kp, feats_t,
                        W0a, b0a.reshape(1, 64), W0b, b0b.reshape(1, 64))

    idxa, idxb = _select(gm3, g2, kpt, k2)

    ga, gb = _SC_GATHER(idxa.reshape(G * NS_A), idxb.reshape(G * NS_B), qab)

    pa = _mlp_max(ga, da, W1a, b1a.reshape(1, 64), NS_A, 0)
    pb = _mlp_max(gb, db, W1b, b1b.reshape(1, 64), NS_B, 1)

    pooled = jnp.concatenate([pa, pb], axis=-1)
    pf = (pooled.reshape(N_PROP, M_GRID, 8, 16)
          .transpose(1, 2, 3, 0).reshape(N_PROP, M_GRID * C_FEAT))
    out = _tail(pf, Wr1, br1.reshape(1, 512), Wr2, br2.reshape(1, 256))
    return out.reshape(1, N_PROP, 256)
